# baseline (speedup 1.0000x reference)
import math

import jax
import jax.numpy as jnp
from jax.experimental import pallas as pl
from jax.experimental.pallas import tpu as pltpu

M = 4096
K = 16
FEAT = 256
BLK = 256
NBLK = M // BLK
L_POS = 10
L_EDGE = 6

_GRIPPER_OFFSETS = [
    [0.0, 0.0, 0.0],
    [0.04, 0.0, 0.0],
    [-0.04, 0.0, 0.0],
    [0.0, 0.0, 0.05],
    [0.04, 0.0, 0.05],
    [-0.04, 0.0, 0.05],
]

_HIGH = jax.lax.Precision.HIGHEST


def _sincos(coords, freqs):
    pieces_s = []
    pieces_c = []
    for d in range(3):
        scaled = (coords[:, d:d + 1] * freqs) * math.pi
        pieces_s.append(jnp.sin(scaled))
        pieces_c.append(jnp.cos(scaled))
    return jnp.concatenate(pieces_s, axis=1), jnp.concatenate(pieces_c, axis=1)


def _kernel(gs_ref,
            pc_blk_ref, pcT_ref, pf_ref, Wp_ref, bp_ref, We_ref, be_ref,
            emb_type_ref, emb_state_ref, R_ref, t_ref, goff_ref,
            f10_ref, f6_ref,
            pcx_ref, edge_ref, gx_ref,
            d2_ref, x_ref):
    i = pl.program_id(0)

    pc_blk = pc_blk_ref[...]
    pcT = pcT_ref[...]
    f10 = f10_ref[...]
    f6 = f6_ref[...]

    s30, c30 = _sincos(pc_blk, f10)
    x_pc = jnp.concatenate([pc_blk, s30, c30], axis=1)
    pcx_ref[...] = (pf_ref[...] + bp_ref[...] +
                    jax.lax.dot_general(x_pc, Wp_ref[...],
                                        (((1,), (0,)), ((), ())),
                                        preferred_element_type=jnp.float32))

    dot = jax.lax.dot_general(pc_blk.astype(jnp.bfloat16),
                              pcT.astype(jnp.bfloat16),
                              (((1,), (0,)), ((), ())),
                              preferred_element_type=jnp.float32)
    sq_c = jnp.sum(pcT * pcT, axis=0, keepdims=True)
    sq_r = jnp.sum(pc_blk * pc_blk, axis=1, keepdims=True)
    d2 = sq_r + sq_c - 2.0 * dot

    iota_c = jax.lax.broadcasted_iota(jnp.int32, (BLK, M), 1)
    row_ids = i * BLK + jax.lax.broadcasted_iota(jnp.int32, (BLK, 1), 0)
    d2_ref[...] = jnp.where(iota_c == row_ids, d2 + 1e10, d2)

    m0 = jnp.min(d2_ref[...], axis=1, keepdims=True)

    def body(k, m):
        d2 = d2_ref[...]
        iota_c = jax.lax.broadcasted_iota(jnp.int32, (BLK, M), 1)
        masked_iota = jnp.where(d2 == m, iota_c, M)
        amin = jnp.min(masked_iota, axis=1, keepdims=True)
        eqmin = iota_c == amin
        sel = jnp.concatenate(
            [jnp.min(jnp.where(eqmin, pcT[d:d + 1, :], 1e30), axis=1,
                     keepdims=True) for d in range(3)], axis=1)
        d2_masked = jnp.where(eqmin, 1e10, d2)
        d2_ref[...] = d2_masked
        m_next = jnp.min(d2_masked, axis=1, keepdims=True)
        delta = pc_blk - sel
        dist = jnp.sqrt(jnp.sum(delta * delta, axis=1, keepdims=True))
        s18, c18 = _sincos(delta, f6)
        x_ref[pl.ds(k * BLK, BLK), :] = jnp.concatenate(
            [delta, s18, c18, dist], axis=1)
        return m_next

    jax.lax.fori_loop(0, K, body, m0)
    x_all = x_ref[...]
    edge_km = (jax.lax.dot_general(x_all, We_ref[...],
                                   (((1,), (0,)), ((), ())),
                                   preferred_element_type=jnp.float32) +
               be_ref[...])
    for k in range(K):
        edge_ref[:, k, :] = edge_km[k * BLK:(k + 1) * BLK, :]

    gpos = jax.lax.dot_general(goff_ref[...], R_ref[...],
                               (((1,), (1,)), ((), ())),
                               precision=_HIGH,
                               preferred_element_type=jnp.float32) + t_ref[...]
    gs30, gc30 = _sincos(gpos, f10)
    x_g = jnp.concatenate([gpos, gs30, gc30], axis=1)
    est = emb_state_ref[pl.ds(gs_ref[0], 1), :]
    gx_ref[...] = (emb_type_ref[...] + est + bp_ref[...] +
                   jax.lax.dot_general(x_g, Wp_ref[...],
                                       (((1,), (0,)), ((), ())),
                                       preferred_element_type=jnp.float32))


def kernel(point_cloud, point_features, gripper_pose, gripper_state,
           W_pos, b_pos, W_edge, b_edge, emb_type, emb_state):
    f32 = jnp.float32
    pcT = point_cloud.T

    def rearrange(W, L):
        Wd = W[0:3]
        idx_sin = [3 + d * 2 * L + 2 * f for d in range(3) for f in range(L)]
        idx_cos = [3 + d * 2 * L + 2 * f + 1 for d in range(3) for f in range(L)]
        return jnp.concatenate([Wd, W[jnp.array(idx_sin)], W[jnp.array(idx_cos)]], axis=0)

    Wp_re = rearrange(W_pos, L_POS)
    We_re = jnp.concatenate([rearrange(W_edge[:-1], L_EDGE), W_edge[-1:]], axis=0)

    R = gripper_pose[:3, :3]
    t = gripper_pose[:3, 3].reshape(1, 3)
    goff = jnp.array(_GRIPPER_OFFSETS, dtype=f32)
    f10 = (2.0 ** jnp.arange(L_POS, dtype=f32)).reshape(1, L_POS)
    f6 = (2.0 ** jnp.arange(L_EDGE, dtype=f32)).reshape(1, L_EDGE)
    gs = gripper_state.astype(jnp.int32)

    full = lambda shape: pl.BlockSpec(shape, lambda i: tuple(0 for _ in shape))
    grid_spec = pltpu.PrefetchScalarGridSpec(
        num_scalar_prefetch=1,
        grid=(NBLK,),
        in_specs=[
            pl.BlockSpec((BLK, 3), lambda i, gs_ref: (i, 0)),
            pl.BlockSpec((3, M), lambda i, gs_ref: (0, 0)),
            pl.BlockSpec((BLK, FEAT), lambda i, gs_ref: (i, 0)),
            pl.BlockSpec((63, FEAT), lambda i, gs_ref: (0, 0)),
            pl.BlockSpec((1, FEAT), lambda i, gs_ref: (0, 0)),
            pl.BlockSpec((40, FEAT), lambda i, gs_ref: (0, 0)),
            pl.BlockSpec((1, FEAT), lambda i, gs_ref: (0, 0)),
            pl.BlockSpec((6, FEAT), lambda i, gs_ref: (0, 0)),
            pl.BlockSpec((2, FEAT), lambda i, gs_ref: (0, 0)),
            pl.BlockSpec((3, 3), lambda i, gs_ref: (0, 0)),
            pl.BlockSpec((1, 3), lambda i, gs_ref: (0, 0)),
            pl.BlockSpec((6, 3), lambda i, gs_ref: (0, 0)),
            pl.BlockSpec((1, L_POS), lambda i, gs_ref: (0, 0)),
            pl.BlockSpec((1, L_EDGE), lambda i, gs_ref: (0, 0)),
        ],
        out_specs=[
            pl.BlockSpec((BLK, FEAT), lambda i, gs_ref: (i, 0)),
            pl.BlockSpec((BLK, K, FEAT), lambda i, gs_ref: (i, 0, 0)),
            pl.BlockSpec((6, FEAT), lambda i, gs_ref: (0, 0)),
        ],
        scratch_shapes=[
            pltpu.VMEM((BLK, M), jnp.float32),
            pltpu.VMEM((K * BLK, 40), jnp.float32),
        ],
    )

    pc_x, edge3d, g_x = pl.pallas_call(
        _kernel,
        grid_spec=grid_spec,
        out_shape=[
            jax.ShapeDtypeStruct((M, FEAT), f32),
            jax.ShapeDtypeStruct((M, K, FEAT), f32),
            jax.ShapeDtypeStruct((6, FEAT), f32),
        ],
    )(gs, point_cloud, pcT, point_features, Wp_re, b_pos.reshape(1, FEAT),
      We_re, b_edge.reshape(1, FEAT), emb_type, emb_state, R, t, goff, f10, f6)

    return jnp.concatenate([pc_x, edge3d.reshape(M * K, FEAT), g_x], axis=0)


# speedup vs baseline: 1.0814x; 1.0814x over previous
import math

import jax
import jax.numpy as jnp
from jax.experimental import pallas as pl
from jax.experimental.pallas import tpu as pltpu

M = 4096
K = 16
FEAT = 256
BLK = 512
NBLK = M // BLK
L_POS = 10
L_EDGE = 6

_GRIPPER_OFFSETS = [
    [0.0, 0.0, 0.0],
    [0.04, 0.0, 0.0],
    [-0.04, 0.0, 0.0],
    [0.0, 0.0, 0.05],
    [0.04, 0.0, 0.05],
    [-0.04, 0.0, 0.05],
]

_HIGH = jax.lax.Precision.HIGHEST


def _sincos(coords, freqs):
    pieces_s = []
    pieces_c = []
    for d in range(3):
        scaled = (coords[:, d:d + 1] * freqs) * math.pi
        pieces_s.append(jnp.sin(scaled))
        pieces_c.append(jnp.cos(scaled))
    return jnp.concatenate(pieces_s, axis=1), jnp.concatenate(pieces_c, axis=1)


def _kernel(gs_ref,
            pc_blk_ref, pcT_ref, pf_ref, Wp_ref, bp_ref, We_ref, be_ref,
            emb_type_ref, emb_state_ref, R_ref, t_ref, goff_ref,
            f10_ref, f6_ref,
            pcx_ref, edge_ref, gx_ref,
            d2_ref, x_ref):
    i = pl.program_id(0)

    pc_blk = pc_blk_ref[...]
    pcT = pcT_ref[...]
    f10 = f10_ref[...]
    f6 = f6_ref[...]

    s30, c30 = _sincos(pc_blk, f10)
    x_pc = jnp.concatenate([pc_blk, s30, c30], axis=1)
    pcx_ref[...] = (pf_ref[...] + bp_ref[...] +
                    jax.lax.dot_general(x_pc, Wp_ref[...],
                                        (((1,), (0,)), ((), ())),
                                        preferred_element_type=jnp.float32))

    dot = jax.lax.dot_general(pc_blk.astype(jnp.bfloat16),
                              pcT.astype(jnp.bfloat16),
                              (((1,), (0,)), ((), ())),
                              preferred_element_type=jnp.float32)
    sq_c = jnp.sum(pcT * pcT, axis=0, keepdims=True)
    sq_r = jnp.sum(pc_blk * pc_blk, axis=1, keepdims=True)
    d2 = sq_r + sq_c - 2.0 * dot

    iota_c = jax.lax.broadcasted_iota(jnp.int32, (BLK, M), 1)
    row_ids = i * BLK + jax.lax.broadcasted_iota(jnp.int32, (BLK, 1), 0)
    d2_ref[...] = jnp.where(iota_c == row_ids, d2 + 1e10, d2)

    def body(k, _):
        d2 = d2_ref[...]
        iota_c = jax.lax.broadcasted_iota(jnp.int32, (BLK, M), 1)
        m = jnp.min(d2, axis=1, keepdims=True)
        masked_iota = jnp.where(d2 == m, iota_c, M)
        amin = jnp.min(masked_iota, axis=1, keepdims=True)
        eqmin = iota_c == amin
        sel = jnp.concatenate(
            [jnp.min(jnp.where(eqmin, pcT[d:d + 1, :], 1e30), axis=1,
                     keepdims=True) for d in range(3)], axis=1)
        d2_ref[...] = jnp.where(eqmin, 1e10, d2)
        delta = pc_blk - sel
        dist = jnp.sqrt(jnp.sum(delta * delta, axis=1, keepdims=True))
        s18, c18 = _sincos(delta, f6)
        x_ref[pl.ds(k * BLK, BLK), :] = jnp.concatenate(
            [delta, s18, c18, dist], axis=1)
        return 0

    jax.lax.fori_loop(0, K, body, 0)
    x_all = x_ref[...]
    edge_km = (jax.lax.dot_general(x_all, We_ref[...],
                                   (((1,), (0,)), ((), ())),
                                   preferred_element_type=jnp.float32) +
               be_ref[...])
    for k in range(K):
        edge_ref[:, k, :] = edge_km[k * BLK:(k + 1) * BLK, :]

    gpos = jax.lax.dot_general(goff_ref[...], R_ref[...],
                               (((1,), (1,)), ((), ())),
                               precision=_HIGH,
                               preferred_element_type=jnp.float32) + t_ref[...]
    gs30, gc30 = _sincos(gpos, f10)
    x_g = jnp.concatenate([gpos, gs30, gc30], axis=1)
    est = emb_state_ref[pl.ds(gs_ref[0], 1), :]
    gx_ref[...] = (emb_type_ref[...] + est + bp_ref[...] +
                   jax.lax.dot_general(x_g, Wp_ref[...],
                                       (((1,), (0,)), ((), ())),
                                       preferred_element_type=jnp.float32))


def kernel(point_cloud, point_features, gripper_pose, gripper_state,
           W_pos, b_pos, W_edge, b_edge, emb_type, emb_state):
    f32 = jnp.float32
    pcT = point_cloud.T

    def rearrange(W, L):
        Wd = W[0:3]
        idx_sin = [3 + d * 2 * L + 2 * f for d in range(3) for f in range(L)]
        idx_cos = [3 + d * 2 * L + 2 * f + 1 for d in range(3) for f in range(L)]
        return jnp.concatenate([Wd, W[jnp.array(idx_sin)], W[jnp.array(idx_cos)]], axis=0)

    Wp_re = rearrange(W_pos, L_POS)
    We_re = jnp.concatenate([rearrange(W_edge[:-1], L_EDGE), W_edge[-1:]], axis=0)

    R = gripper_pose[:3, :3]
    t = gripper_pose[:3, 3].reshape(1, 3)
    goff = jnp.array(_GRIPPER_OFFSETS, dtype=f32)
    f10 = (2.0 ** jnp.arange(L_POS, dtype=f32)).reshape(1, L_POS)
    f6 = (2.0 ** jnp.arange(L_EDGE, dtype=f32)).reshape(1, L_EDGE)
    gs = gripper_state.astype(jnp.int32)

    full = lambda shape: pl.BlockSpec(shape, lambda i: tuple(0 for _ in shape))
    grid_spec = pltpu.PrefetchScalarGridSpec(
        num_scalar_prefetch=1,
        grid=(NBLK,),
        in_specs=[
            pl.BlockSpec((BLK, 3), lambda i, gs_ref: (i, 0)),
            pl.BlockSpec((3, M), lambda i, gs_ref: (0, 0)),
            pl.BlockSpec((BLK, FEAT), lambda i, gs_ref: (i, 0)),
            pl.BlockSpec((63, FEAT), lambda i, gs_ref: (0, 0)),
            pl.BlockSpec((1, FEAT), lambda i, gs_ref: (0, 0)),
            pl.BlockSpec((40, FEAT), lambda i, gs_ref: (0, 0)),
            pl.BlockSpec((1, FEAT), lambda i, gs_ref: (0, 0)),
            pl.BlockSpec((6, FEAT), lambda i, gs_ref: (0, 0)),
            pl.BlockSpec((2, FEAT), lambda i, gs_ref: (0, 0)),
            pl.BlockSpec((3, 3), lambda i, gs_ref: (0, 0)),
            pl.BlockSpec((1, 3), lambda i, gs_ref: (0, 0)),
            pl.BlockSpec((6, 3), lambda i, gs_ref: (0, 0)),
            pl.BlockSpec((1, L_POS), lambda i, gs_ref: (0, 0)),
            pl.BlockSpec((1, L_EDGE), lambda i, gs_ref: (0, 0)),
        ],
        out_specs=[
            pl.BlockSpec((BLK, FEAT), lambda i, gs_ref: (i, 0)),
            pl.BlockSpec((BLK, K, FEAT), lambda i, gs_ref: (i, 0, 0)),
            pl.BlockSpec((6, FEAT), lambda i, gs_ref: (0, 0)),
        ],
        scratch_shapes=[
            pltpu.VMEM((BLK, M), jnp.float32),
            pltpu.VMEM((K * BLK, 40), jnp.float32),
        ],
    )

    pc_x, edge3d, g_x = pl.pallas_call(
        _kernel,
        grid_spec=grid_spec,
        out_shape=[
            jax.ShapeDtypeStruct((M, FEAT), f32),
            jax.ShapeDtypeStruct((M, K, FEAT), f32),
            jax.ShapeDtypeStruct((6, FEAT), f32),
        ],
    )(gs, point_cloud, pcT, point_features, Wp_re, b_pos.reshape(1, FEAT),
      We_re, b_edge.reshape(1, FEAT), emb_type, emb_state, R, t, goff, f10, f6)

    return jnp.concatenate([pc_x, edge3d.reshape(M * K, FEAT), g_x], axis=0)
